# Initial kernel scaffold; baseline (speedup 1.0000x reference)
import functools

import jax
import jax.numpy as jnp
from jax import lax
from jax.experimental import pallas as pl
from jax.experimental.pallas import tpu as pltpu
from jax.experimental.pallas import tpu_sc as plsc

_NC = 2
_NS = 16
_NW = _NC * _NS


def _gather_body(nchunks, chunk, table_hbm, idx_hbm, out_hbm, idx_v, rows_v, sem):
    per_w = nchunks * chunk
    wid = lax.axis_index("s") * _NC + lax.axis_index("c")
    base = wid * per_w
    for i in range(nchunks):
        off = base + i * chunk
        pltpu.sync_copy(idx_hbm.at[pl.ds(off, chunk)], idx_v)
        pltpu.async_copy(table_hbm.at[idx_v], rows_v, sem).wait()
        pltpu.sync_copy(rows_v, out_hbm.at[pl.ds(off, chunk)])


def _sc_gather(table_flat, flat_idx):
    n, d = flat_idx.shape[0], table_flat.shape[1]
    per_w = n // _NW
    nchunks = 8
    chunk = per_w // nchunks
    mesh = plsc.VectorSubcoreMesh(
        core_axis_name="c", subcore_axis_name="s",
        num_cores=_NC, num_subcores=_NS)
    return pl.kernel(
        functools.partial(_gather_body, nchunks, chunk),
        out_type=jax.ShapeDtypeStruct((n, d), jnp.float32),
        mesh=mesh,
        scratch_types=[
            pltpu.VMEM((chunk,), jnp.int32),
            pltpu.VMEM((chunk, d), jnp.float32),
            pltpu.SemaphoreType.DMA,
        ],
    )(table_flat, flat_idx)


def _mlp_body(emb_ref, den_ref, w1s_ref, w1d_ref, b1_ref, w2_ref, b2_ref,
              w3_ref, b3_ref, ww_ref, bw_ref, wf_ref, bf_ref, out_ref):
    x = emb_ref[...]
    d = den_ref[...]
    h = jnp.dot(x, w1s_ref[...], preferred_element_type=jnp.float32)
    h += jnp.dot(d, w1d_ref[...], preferred_element_type=jnp.float32)
    h = jnp.maximum(h + b1_ref[...], 0.0)
    h = jnp.maximum(
        jnp.dot(h, w2_ref[...], preferred_element_type=jnp.float32) + b2_ref[...], 0.0)
    h = jnp.maximum(
        jnp.dot(h, w3_ref[...], preferred_element_type=jnp.float32) + b3_ref[...], 0.0)
    deep = jnp.dot(h, wf_ref[...], preferred_element_type=jnp.float32) + bf_ref[...]
    wide = jnp.dot(d, ww_ref[...], preferred_element_type=jnp.float32) + bw_ref[...]
    out_ref[...] = jax.nn.sigmoid(0.5 * wide + 0.5 * deep)


def _tc_mlp(emb2d, dense, W1s, W1d, b1, W2, b2, W3, b3, Ww, bw, Wf, bf):
    b, fd = emb2d.shape
    nd = dense.shape[1]
    blk = 2048
    grid = (b // blk,)
    full = lambda a: pl.BlockSpec(a.shape, lambda i: (0, 0))
    return pl.pallas_call(
        _mlp_body,
        grid=grid,
        in_specs=[
            pl.BlockSpec((blk, fd), lambda i: (i, 0)),
            pl.BlockSpec((blk, nd), lambda i: (i, 0)),
            full(W1s), full(W1d), full(b1), full(W2), full(b2),
            full(W3), full(b3), full(Ww), full(bw), full(Wf), full(bf),
        ],
        out_specs=pl.BlockSpec((blk, 1), lambda i: (i, 0)),
        out_shape=jax.ShapeDtypeStruct((b, 1), jnp.float32),
    )(emb2d, dense, W1s, W1d, b1, W2, b2, W3, b3, Ww, bw, Wf, bf)


def kernel(dense_inputs, sparse_inputs, tables, W1, b1, W2, b2, W3, b3,
           W_wide, b_wide, W_final, b_final):
    b, f = sparse_inputs.shape
    v, d = tables.shape[1], tables.shape[2]
    flat_idx = (sparse_inputs
                + (jnp.arange(f, dtype=jnp.int32) * v)[None, :]).reshape(b * f)
    table_flat = tables.reshape(f * v, d)
    emb = _sc_gather(table_flat, flat_idx)
    emb2d = emb.reshape(b, f * d)
    W1s, W1d = W1[: f * d], W1[f * d:]
    return _tc_mlp(
        emb2d, dense_inputs, W1s, W1d, b1.reshape(1, -1),
        W2, b2.reshape(1, -1), W3, b3.reshape(1, -1),
        W_wide, b_wide.reshape(1, -1), W_final, b_final.reshape(1, -1))


# speedup vs baseline: 7.8177x; 7.8177x over previous
import functools

import jax
import jax.numpy as jnp
from jax import lax
from jax.experimental import pallas as pl
from jax.experimental.pallas import tpu as pltpu
from jax.experimental.pallas import tpu_sc as plsc

_NC = 2
_NS = 16
_NW = _NC * _NS


def _gather_body(nchunks, chunk, table_hbm, idx_hbm, out_hbm, idx_v, rows_v, sem):
    per_w = nchunks * chunk
    wid = lax.axis_index("s") * _NC + lax.axis_index("c")
    base = wid * per_w
    for i in range(nchunks):
        off = base + i * chunk
        pltpu.sync_copy(idx_hbm.at[pl.ds(off, chunk)], idx_v)
        pltpu.async_copy(table_hbm.at[idx_v], rows_v, sem).wait()
        pltpu.sync_copy(rows_v, out_hbm.at[pl.ds(off, chunk)])


def _sc_gather(table_flat, flat_idx):
    n, d = flat_idx.shape[0], table_flat.shape[1]
    per_w = n // _NW
    nchunks = 8
    chunk = per_w // nchunks
    mesh = plsc.VectorSubcoreMesh(
        core_axis_name="c", subcore_axis_name="s",
        num_cores=_NC, num_subcores=_NS)
    return pl.kernel(
        functools.partial(_gather_body, nchunks, chunk),
        out_type=jax.ShapeDtypeStruct((n, d), jnp.float32),
        mesh=mesh,
        scratch_types=[
            pltpu.VMEM((chunk,), jnp.int32),
            pltpu.VMEM((chunk, d), jnp.float32),
            pltpu.SemaphoreType.DMA,
        ],
        compiler_params=pltpu.CompilerParams(use_tc_tiling_on_sc=False),
    )(table_flat, flat_idx)


def _mlp_body(emb_ref, den_ref, w1s_ref, w1d_ref, b1_ref, w2_ref, b2_ref,
              w3_ref, b3_ref, ww_ref, bw_ref, wf_ref, bf_ref, out_ref):
    x = emb_ref[...]
    d = den_ref[...]
    h = jnp.dot(x, w1s_ref[...], preferred_element_type=jnp.float32)
    h += jnp.dot(d, w1d_ref[...], preferred_element_type=jnp.float32)
    h = jnp.maximum(h + b1_ref[...], 0.0)
    h = jnp.maximum(
        jnp.dot(h, w2_ref[...], preferred_element_type=jnp.float32) + b2_ref[...], 0.0)
    h = jnp.maximum(
        jnp.dot(h, w3_ref[...], preferred_element_type=jnp.float32) + b3_ref[...], 0.0)
    deep = jnp.dot(h, wf_ref[...], preferred_element_type=jnp.float32) + bf_ref[...]
    wide = jnp.dot(d, ww_ref[...], preferred_element_type=jnp.float32) + bw_ref[...]
    out_ref[...] = jax.nn.sigmoid(0.5 * wide + 0.5 * deep)


def _tc_mlp(emb2d, dense, W1s, W1d, b1, W2, b2, W3, b3, Ww, bw, Wf, bf):
    b, fd = emb2d.shape
    nd = dense.shape[1]
    blk = 2048
    grid = (b // blk,)
    full = lambda a: pl.BlockSpec(a.shape, lambda i: (0, 0))
    return pl.pallas_call(
        _mlp_body,
        grid=grid,
        in_specs=[
            pl.BlockSpec((blk, fd), lambda i: (i, 0)),
            pl.BlockSpec((blk, nd), lambda i: (i, 0)),
            full(W1s), full(W1d), full(b1), full(W2), full(b2),
            full(W3), full(b3), full(Ww), full(bw), full(Wf), full(bf),
        ],
        out_specs=pl.BlockSpec((blk, 1), lambda i: (i, 0)),
        out_shape=jax.ShapeDtypeStruct((b, 1), jnp.float32),
    )(emb2d, dense, W1s, W1d, b1, W2, b2, W3, b3, Ww, bw, Wf, bf)


def kernel(dense_inputs, sparse_inputs, tables, W1, b1, W2, b2, W3, b3,
           W_wide, b_wide, W_final, b_final):
    b, f = sparse_inputs.shape
    v, d = tables.shape[1], tables.shape[2]
    flat_idx = (sparse_inputs
                + (jnp.arange(f, dtype=jnp.int32) * v)[None, :]).reshape(b * f)
    table_flat = tables.reshape(f * v, d)
    emb = _sc_gather(table_flat, flat_idx)
    emb2d = emb.reshape(b, f * d)
    W1s, W1d = W1[: f * d], W1[f * d:]
    return _tc_mlp(
        emb2d, dense_inputs, W1s, W1d, b1.reshape(1, -1),
        W2, b2.reshape(1, -1), W3, b3.reshape(1, -1),
        W_wide, b_wide.reshape(1, -1), W_final, b_final.reshape(1, -1))
